# baseline (speedup 1.0000x reference)
import functools

import jax
import jax.numpy as jnp
from jax import lax
from jax.experimental import pallas as pl

N = 32768
F = 128
Z = 128
K = 256
TILES = 16
TILE_ROWS = N // TILES


def _scores_body(bags_ref, wenc_ref, benc_ref, watt_ref, out_ref):
    emb = jnp.maximum(
        jnp.dot(bags_ref[...], wenc_ref[...], preferred_element_type=jnp.float32)
        + benc_ref[...],
        0.0,
    )
    s = lax.dot_general(
        watt_ref[...], emb, (((1,), (1,)), ((), ())),
        preferred_element_type=jnp.float32,
    )
    out_ref[...] = s.reshape(1, 1, TILE_ROWS)


_scores_call = pl.pallas_call(
    _scores_body,
    grid=(TILES,),
    in_specs=[
        pl.BlockSpec((TILE_ROWS, F), lambda i: (i, 0)),
        pl.BlockSpec((F, Z), lambda i: (0, 0)),
        pl.BlockSpec((1, Z), lambda i: (0, 0)),
        pl.BlockSpec((1, Z), lambda i: (0, 0)),
    ],
    out_specs=pl.BlockSpec((1, 1, TILE_ROWS), lambda i: (i, 0, 0)),
    out_shape=jax.ShapeDtypeStruct((TILES, 1, TILE_ROWS), jnp.float32),
)


def _head_body(rows_ref, wenc_ref, benc_ref, gamma_ref, beta_ref, mean_ref,
               var_ref, whead_ref, bhead_ref, out_ref):
    emb = jnp.maximum(
        jnp.dot(rows_ref[...], wenc_ref[...], preferred_element_type=jnp.float32)
        + benc_ref[...],
        0.0,
    )
    pooled = jnp.sum(emb, axis=0, keepdims=True) * (1.0 / K)
    h = (pooled - mean_ref[...]) * lax.rsqrt(var_ref[...] + 1e-5) * gamma_ref[...] + beta_ref[...]
    out_ref[...] = jnp.dot(h, whead_ref[...], preferred_element_type=jnp.float32) + bhead_ref[...]


_head_call = pl.pallas_call(
    _head_body,
    in_specs=[
        pl.BlockSpec((K, F), lambda: (0, 0)),
        pl.BlockSpec((F, Z), lambda: (0, 0)),
        pl.BlockSpec((1, Z), lambda: (0, 0)),
        pl.BlockSpec((1, Z), lambda: (0, 0)),
        pl.BlockSpec((1, Z), lambda: (0, 0)),
        pl.BlockSpec((1, Z), lambda: (0, 0)),
        pl.BlockSpec((1, Z), lambda: (0, 0)),
        pl.BlockSpec((Z, 2), lambda: (0, 0)),
        pl.BlockSpec((1, 2), lambda: (0, 0)),
    ],
    out_specs=pl.BlockSpec((1, 2), lambda: (0, 0)),
    out_shape=jax.ShapeDtypeStruct((1, 2), jnp.float32),
)


def kernel(bags, W_enc, b_enc, W_att, b_att, bn_gamma, bn_beta, bn_mean,
           bn_var, W_head, b_head):
    benc2 = b_enc.reshape(1, Z)
    watt2 = W_att.reshape(1, Z)
    scores = _scores_call(bags, W_enc, benc2, watt2).reshape(N) + b_att[0]
    _, topk_idx = lax.top_k(scores, K)
    rows = jnp.take(bags, topk_idx, axis=0)
    out = _head_call(
        rows, W_enc, benc2,
        bn_gamma.reshape(1, Z), bn_beta.reshape(1, Z),
        bn_mean.reshape(1, Z), bn_var.reshape(1, Z),
        W_head, b_head.reshape(1, 2),
    )
    return out.reshape(2)
